# Initial kernel scaffold; baseline (speedup 1.0000x reference)
import jax
import jax.numpy as jnp
from jax.experimental import pallas as pl


def kernel(node_z, node_feats, senders, receivers, edge_weight, edge_feats, emb_table, W_dist, b_dist, W_comb, b_comb):
    raise NotImplementedError("write your pallas kernel here")


# speedup vs baseline: 3.2527x; 3.2527x over previous
import functools

import jax
import jax.numpy as jnp
from jax import lax
from jax.experimental import pallas as pl
from jax.experimental.pallas import tpu as pltpu
from jax.experimental.pallas import tpu_sc as plsc

N_NODES = 10000
N_EDGES = 320000
D = 128
D_EDGE = 16
NUM_SPECIES = 100
CUTOFF = 5.0

NC = 2
NS = 16
NW = NC * NS
EPW = N_EDGES // NW
CH = 80
NCH = EPW // CH
ROW_CH = 80
N_ROW_CH = N_NODES // ROW_CH

E_BLK = 1600
N_EBLK = N_EDGES // E_BLK

NODE_BLK = 1000
N_NBLK = N_NODES // NODE_BLK


def _sc_mesh():
    return plsc.VectorSubcoreMesh(
        core_axis_name="c", subcore_axis_name="s", num_cores=NC,
        num_subcores=NS)


def _zj_body(node_z_hbm, senders_hbm, zj_hbm, nz_v, snd_v, out_v):
    c = lax.axis_index("c")
    s = lax.axis_index("s")
    wid = c * NS + s
    base = wid * EPW
    pltpu.sync_copy(node_z_hbm, nz_v)
    pltpu.sync_copy(senders_hbm.at[pl.ds(base, EPW)], snd_v)

    def body(i, _):
        idx = snd_v[pl.ds(i * 16, 16)]
        out_v[pl.ds(i * 16, 16)] = plsc.load_gather(nz_v, [idx])
        return 0

    lax.fori_loop(0, EPW // 16, body, 0)
    pltpu.sync_copy(out_v, zj_hbm.at[pl.ds(base, EPW)])


def _gather_zj(node_z, senders):
    return pl.kernel(
        _zj_body,
        out_type=jax.ShapeDtypeStruct((N_EDGES,), jnp.int32),
        mesh=_sc_mesh(),
        scratch_types=[
            pltpu.VMEM((N_NODES,), jnp.int32),
            pltpu.VMEM((EPW,), jnp.int32),
            pltpu.VMEM((EPW,), jnp.int32),
        ],
        compiler_params=pltpu.CompilerParams(needs_layout_passes=False),
    )(node_z, senders)


def _msg_body(z_ref, ew_ref, ef_ref, emb_ref, wd_ref, bd_ref, m_ref):
    z = z_ref[0, 0]
    ew = ew_ref[0, 0]
    ef = ef_ref[...]
    cvals = 0.5 * (jnp.cos(ew * (jnp.pi / CUTOFF)) + 1.0)
    cvals = jnp.where(ew < CUTOFF, cvals, 0.0)
    onehot = (lax.broadcasted_iota(jnp.int32, (E_BLK, NUM_SPECIES), 1)
              == z[:, None]).astype(jnp.float32)
    xj = jnp.dot(onehot, emb_ref[...], preferred_element_type=jnp.float32)
    p = jnp.dot(ef, wd_ref[...], preferred_element_type=jnp.float32)
    p = p + bd_ref[...]
    m_ref[...] = xj * p * cvals[:, None]


def _messages(z_j, edge_weight, edge_feats, emb_table, W_dist, b_dist):
    return pl.pallas_call(
        _msg_body,
        grid=(N_EBLK,),
        in_specs=[
            pl.BlockSpec((1, 1, E_BLK), lambda i: (i, 0, 0)),
            pl.BlockSpec((1, 1, E_BLK), lambda i: (i, 0, 0)),
            pl.BlockSpec((E_BLK, D_EDGE), lambda i: (i, 0)),
            pl.BlockSpec((NUM_SPECIES, D), lambda i: (0, 0)),
            pl.BlockSpec((D_EDGE, D), lambda i: (0, 0)),
            pl.BlockSpec((1, D), lambda i: (0, 0)),
        ],
        out_specs=pl.BlockSpec((E_BLK, D), lambda i: (i, 0)),
        out_shape=jax.ShapeDtypeStruct((N_EDGES, D), jnp.float32),
    )(z_j.reshape(N_EBLK, 1, E_BLK), edge_weight.reshape(N_EBLK, 1, E_BLK),
      edge_feats, emb_table, W_dist, b_dist.reshape(1, D))


def _seg_body(recv_hbm, m_hbm, out_hbm, agg_sh, ridx_v, m_v):
    c = lax.axis_index("c")
    s = lax.axis_index("s")
    wid = c * NS + s
    base = wid * EPW

    def zrow(i, _):
        for k in range(D // 16):
            m_v[i, pl.ds(k * 16, 16)] = jnp.zeros((16,), jnp.float32)
        return 0

    lax.fori_loop(0, CH, zrow, 0)

    def zchunk(k, _):
        j = s + k * NS

        @pl.when(j < N_ROW_CH)
        def _():
            pltpu.sync_copy(m_v, agg_sh.at[pl.ds(j * ROW_CH, ROW_CH)])
        return 0

    lax.fori_loop(0, (N_ROW_CH + NS - 1) // NS, zchunk, 0)
    plsc.subcore_barrier()

    def chunk(j, _):
        off = base + j * CH
        pltpu.sync_copy(recv_hbm.at[pl.ds(off, CH)], ridx_v)
        pltpu.sync_copy(m_hbm.at[pl.ds(off, CH)], m_v)
        pltpu.sync_copy(m_v, agg_sh.at[ridx_v], add=True)
        return 0

    lax.fori_loop(0, NCH, chunk, 0)
    plsc.subcore_barrier()

    def drain(k, _):
        j = s + k * NS

        @pl.when(j < N_ROW_CH)
        def _():
            pltpu.sync_copy(agg_sh.at[pl.ds(j * ROW_CH, ROW_CH)], m_v)
            pltpu.sync_copy(m_v, out_hbm.at[c, pl.ds(j * ROW_CH, ROW_CH)])
        return 0

    lax.fori_loop(0, (N_ROW_CH + NS - 1) // NS, drain, 0)


def _segment_sum(receivers, m):
    return pl.kernel(
        _seg_body,
        out_type=jax.ShapeDtypeStruct((NC, N_NODES, D), jnp.float32),
        mesh=_sc_mesh(),
        scratch_types=[
            pltpu.VMEM_SHARED((N_NODES, D), jnp.float32),
            pltpu.VMEM((CH,), jnp.int32),
            pltpu.VMEM((CH, D), jnp.float32),
        ],
    )(receivers, m)


def _comb_body(nf_ref, agg_ref, wt_ref, wb_ref, bc_ref, o_ref):
    agg = agg_ref[0] + agg_ref[1]
    o_ref[...] = (
        jnp.dot(nf_ref[...], wt_ref[...], preferred_element_type=jnp.float32)
        + jnp.dot(agg, wb_ref[...], preferred_element_type=jnp.float32)
        + bc_ref[...])


def _combine(node_feats, agg2, W_comb, b_comb):
    return pl.pallas_call(
        _comb_body,
        grid=(N_NBLK,),
        in_specs=[
            pl.BlockSpec((NODE_BLK, D), lambda i: (i, 0)),
            pl.BlockSpec((NC, NODE_BLK, D), lambda i: (0, i, 0)),
            pl.BlockSpec((D, D), lambda i: (0, 0)),
            pl.BlockSpec((D, D), lambda i: (0, 0)),
            pl.BlockSpec((1, D), lambda i: (0, 0)),
        ],
        out_specs=pl.BlockSpec((NODE_BLK, D), lambda i: (i, 0)),
        out_shape=jax.ShapeDtypeStruct((N_NODES, D), jnp.float32),
    )(node_feats, agg2, W_comb[:D], W_comb[D:], b_comb.reshape(1, D))


def kernel(node_z, node_feats, senders, receivers, edge_weight, edge_feats,
           emb_table, W_dist, b_dist, W_comb, b_comb):
    node_z = node_z.astype(jnp.int32)
    senders = senders.astype(jnp.int32)
    receivers = receivers.astype(jnp.int32)
    z_j = _gather_zj(node_z, senders)
    m = _messages(z_j, edge_weight, edge_feats, emb_table, W_dist, b_dist)
    agg2 = _segment_sum(receivers, m)
    return _combine(node_feats, agg2, W_comb, b_comb)


# speedup vs baseline: 3.7403x; 1.1499x over previous
import functools

import jax
import jax.numpy as jnp
from jax import lax
from jax.experimental import pallas as pl
from jax.experimental.pallas import tpu as pltpu
from jax.experimental.pallas import tpu_sc as plsc

N_NODES = 10000
N_EDGES = 320000
D = 128
D_EDGE = 16
NUM_SPECIES = 100
CUTOFF = 5.0

NC = 2
NS = 16
NW = NC * NS
EPW = N_EDGES // NW
CH = 200
NCH = EPW // CH
ROW_CH = 200
N_ROW_CH = N_NODES // ROW_CH

E_BLK = 1600
N_EBLK = N_EDGES // E_BLK

NODE_BLK = 1000
N_NBLK = N_NODES // NODE_BLK


def _sc_mesh():
    return plsc.VectorSubcoreMesh(
        core_axis_name="c", subcore_axis_name="s", num_cores=NC,
        num_subcores=NS)


def _zj_body(node_z_hbm, senders_hbm, zj_hbm, nz_v, snd_v, out_v):
    c = lax.axis_index("c")
    s = lax.axis_index("s")
    wid = c * NS + s
    base = wid * EPW
    pltpu.sync_copy(node_z_hbm, nz_v)
    pltpu.sync_copy(senders_hbm.at[pl.ds(base, EPW)], snd_v)

    def body(i, _):
        idx = snd_v[pl.ds(i * 16, 16)]
        out_v[pl.ds(i * 16, 16)] = plsc.load_gather(nz_v, [idx])
        return 0

    lax.fori_loop(0, EPW // 16, body, 0)
    pltpu.sync_copy(out_v, zj_hbm.at[pl.ds(base, EPW)])


def _gather_zj(node_z, senders):
    return pl.kernel(
        _zj_body,
        out_type=jax.ShapeDtypeStruct((N_EDGES,), jnp.int32),
        mesh=_sc_mesh(),
        scratch_types=[
            pltpu.VMEM((N_NODES,), jnp.int32),
            pltpu.VMEM((EPW,), jnp.int32),
            pltpu.VMEM((EPW,), jnp.int32),
        ],
        compiler_params=pltpu.CompilerParams(needs_layout_passes=False),
    )(node_z, senders)


def _msg_body(z_ref, ew_ref, ef_ref, emb_ref, wd_ref, bd_ref, m_ref):
    z = z_ref[0, 0]
    ew = ew_ref[0, 0]
    ef = ef_ref[...]
    cvals = 0.5 * (jnp.cos(ew * (jnp.pi / CUTOFF)) + 1.0)
    cvals = jnp.where(ew < CUTOFF, cvals, 0.0)
    onehot = (lax.broadcasted_iota(jnp.int32, (E_BLK, NUM_SPECIES), 1)
              == z[:, None]).astype(jnp.float32)
    xj = jnp.dot(onehot, emb_ref[...], preferred_element_type=jnp.float32)
    p = jnp.dot(ef, wd_ref[...], preferred_element_type=jnp.float32)
    p = p + bd_ref[...]
    m_ref[...] = xj * p * cvals[:, None]


def _messages(z_j, edge_weight, edge_feats, emb_table, W_dist, b_dist):
    return pl.pallas_call(
        _msg_body,
        grid=(N_EBLK,),
        in_specs=[
            pl.BlockSpec((1, 1, E_BLK), lambda i: (i, 0, 0)),
            pl.BlockSpec((1, 1, E_BLK), lambda i: (i, 0, 0)),
            pl.BlockSpec((E_BLK, D_EDGE), lambda i: (i, 0)),
            pl.BlockSpec((NUM_SPECIES, D), lambda i: (0, 0)),
            pl.BlockSpec((D_EDGE, D), lambda i: (0, 0)),
            pl.BlockSpec((1, D), lambda i: (0, 0)),
        ],
        out_specs=pl.BlockSpec((E_BLK, D), lambda i: (i, 0)),
        out_shape=jax.ShapeDtypeStruct((N_EDGES, D), jnp.float32),
    )(z_j.reshape(N_EBLK, 1, E_BLK), edge_weight.reshape(N_EBLK, 1, E_BLK),
      edge_feats, emb_table, W_dist, b_dist.reshape(1, D))


def _seg_body(recv_hbm, m_hbm, out_hbm, agg_sh, ridx_v, m_v):
    c = lax.axis_index("c")
    s = lax.axis_index("s")
    wid = c * NS + s
    base = wid * EPW

    def zrow(i, _):
        for k in range(D // 16):
            m_v[i, pl.ds(k * 16, 16)] = jnp.zeros((16,), jnp.float32)
        return 0

    lax.fori_loop(0, CH, zrow, 0)

    def zchunk(k, _):
        j = s + k * NS

        @pl.when(j < N_ROW_CH)
        def _():
            pltpu.sync_copy(m_v, agg_sh.at[pl.ds(j * ROW_CH, ROW_CH)])
        return 0

    lax.fori_loop(0, (N_ROW_CH + NS - 1) // NS, zchunk, 0)
    plsc.subcore_barrier()

    def chunk(j, _):
        off = base + j * CH
        pltpu.sync_copy(recv_hbm.at[pl.ds(off, CH)], ridx_v)
        pltpu.sync_copy(m_hbm.at[pl.ds(off, CH)], m_v)
        pltpu.sync_copy(m_v, agg_sh.at[ridx_v], add=True)
        return 0

    lax.fori_loop(0, NCH, chunk, 0)
    plsc.subcore_barrier()

    def drain(k, _):
        j = s + k * NS

        @pl.when(j < N_ROW_CH)
        def _():
            pltpu.sync_copy(agg_sh.at[pl.ds(j * ROW_CH, ROW_CH)], m_v)
            pltpu.sync_copy(m_v, out_hbm.at[c, pl.ds(j * ROW_CH, ROW_CH)])
        return 0

    lax.fori_loop(0, (N_ROW_CH + NS - 1) // NS, drain, 0)


def _segment_sum(receivers, m):
    return pl.kernel(
        _seg_body,
        out_type=jax.ShapeDtypeStruct((NC, N_NODES, D), jnp.float32),
        mesh=_sc_mesh(),
        scratch_types=[
            pltpu.VMEM_SHARED((N_NODES, D), jnp.float32),
            pltpu.VMEM((CH,), jnp.int32),
            pltpu.VMEM((CH, D), jnp.float32),
        ],
    )(receivers, m)


def _comb_body(nf_ref, agg_ref, wt_ref, wb_ref, bc_ref, o_ref):
    agg = agg_ref[0] + agg_ref[1]
    o_ref[...] = (
        jnp.dot(nf_ref[...], wt_ref[...], preferred_element_type=jnp.float32)
        + jnp.dot(agg, wb_ref[...], preferred_element_type=jnp.float32)
        + bc_ref[...])


def _combine(node_feats, agg2, W_comb, b_comb):
    return pl.pallas_call(
        _comb_body,
        grid=(N_NBLK,),
        in_specs=[
            pl.BlockSpec((NODE_BLK, D), lambda i: (i, 0)),
            pl.BlockSpec((NC, NODE_BLK, D), lambda i: (0, i, 0)),
            pl.BlockSpec((D, D), lambda i: (0, 0)),
            pl.BlockSpec((D, D), lambda i: (0, 0)),
            pl.BlockSpec((1, D), lambda i: (0, 0)),
        ],
        out_specs=pl.BlockSpec((NODE_BLK, D), lambda i: (i, 0)),
        out_shape=jax.ShapeDtypeStruct((N_NODES, D), jnp.float32),
    )(node_feats, agg2, W_comb[:D], W_comb[D:], b_comb.reshape(1, D))


def kernel(node_z, node_feats, senders, receivers, edge_weight, edge_feats,
           emb_table, W_dist, b_dist, W_comb, b_comb):
    node_z = node_z.astype(jnp.int32)
    senders = senders.astype(jnp.int32)
    receivers = receivers.astype(jnp.int32)
    z_j = _gather_zj(node_z, senders)
    m = _messages(z_j, edge_weight, edge_feats, emb_table, W_dist, b_dist)
    agg2 = _segment_sum(receivers, m)
    return _combine(node_feats, agg2, W_comb, b_comb)


# speedup vs baseline: 4.1142x; 1.1000x over previous
import functools

import jax
import jax.numpy as jnp
from jax import lax
from jax.experimental import pallas as pl
from jax.experimental.pallas import tpu as pltpu
from jax.experimental.pallas import tpu_sc as plsc

N_NODES = 10000
N_EDGES = 320000
D = 128
D_EDGE = 16
NUM_SPECIES = 100
CUTOFF = 5.0

NC = 2
NS = 16
NW = NC * NS
EPW = N_EDGES // NW
NSLICE = 2
E_SLICE = N_EDGES // NSLICE
EPW_S = E_SLICE // NW
CH = 200
NCH = EPW_S // CH
ROW_CH = 200
N_ROW_CH = N_NODES // ROW_CH

E_BLK = 1600
N_EBLK = N_EDGES // E_BLK

NODE_BLK = 1000
N_NBLK = N_NODES // NODE_BLK


def _sc_mesh():
    return plsc.VectorSubcoreMesh(
        core_axis_name="c", subcore_axis_name="s", num_cores=NC,
        num_subcores=NS)


def _zj_body(node_z_hbm, senders_hbm, zj_hbm, nz_v, snd_v, out_v):
    c = lax.axis_index("c")
    s = lax.axis_index("s")
    wid = c * NS + s
    base = wid * EPW
    pltpu.sync_copy(node_z_hbm, nz_v)
    pltpu.sync_copy(senders_hbm.at[pl.ds(base, EPW)], snd_v)

    def body(i, _):
        idx = snd_v[pl.ds(i * 16, 16)]
        out_v[pl.ds(i * 16, 16)] = plsc.load_gather(nz_v, [idx])
        return 0

    lax.fori_loop(0, EPW // 16, body, 0)
    pltpu.sync_copy(out_v, zj_hbm.at[pl.ds(base, EPW)])


def _gather_zj(node_z, senders):
    return pl.kernel(
        _zj_body,
        out_type=jax.ShapeDtypeStruct((N_EDGES,), jnp.int32),
        mesh=_sc_mesh(),
        scratch_types=[
            pltpu.VMEM((N_NODES,), jnp.int32),
            pltpu.VMEM((EPW,), jnp.int32),
            pltpu.VMEM((EPW,), jnp.int32),
        ],
        compiler_params=pltpu.CompilerParams(needs_layout_passes=False),
    )(node_z, senders)


def _msg_body(z_ref, ew_ref, ef_ref, emb_ref, wd_ref, bd_ref, m_ref):
    z = z_ref[0, 0]
    ew = ew_ref[0, 0]
    ef = ef_ref[...]
    cvals = 0.5 * (jnp.cos(ew * (jnp.pi / CUTOFF)) + 1.0)
    cvals = jnp.where(ew < CUTOFF, cvals, 0.0)
    onehot = (lax.broadcasted_iota(jnp.int32, (E_BLK, NUM_SPECIES), 1)
              == z[:, None]).astype(jnp.float32)
    xj = jnp.dot(onehot, emb_ref[...], preferred_element_type=jnp.float32)
    p = jnp.dot(ef, wd_ref[...], preferred_element_type=jnp.float32)
    p = p + bd_ref[...]
    m_ref[...] = xj * p * cvals[:, None]


def _messages(z_j, edge_weight, edge_feats, emb_table, W_dist, b_dist):
    n_blk = z_j.shape[0] // E_BLK
    return pl.pallas_call(
        _msg_body,
        grid=(n_blk,),
        in_specs=[
            pl.BlockSpec((1, 1, E_BLK), lambda i: (i, 0, 0)),
            pl.BlockSpec((1, 1, E_BLK), lambda i: (i, 0, 0)),
            pl.BlockSpec((E_BLK, D_EDGE), lambda i: (i, 0)),
            pl.BlockSpec((NUM_SPECIES, D), lambda i: (0, 0)),
            pl.BlockSpec((D_EDGE, D), lambda i: (0, 0)),
            pl.BlockSpec((1, D), lambda i: (0, 0)),
        ],
        out_specs=pl.BlockSpec((E_BLK, D), lambda i: (i, 0)),
        out_shape=jax.ShapeDtypeStruct((z_j.shape[0], D), jnp.float32),
    )(z_j.reshape(n_blk, 1, E_BLK), edge_weight.reshape(n_blk, 1, E_BLK),
      edge_feats, emb_table, W_dist, b_dist.reshape(1, D))


def _seg_body(recv_hbm, m_hbm, out_hbm, agg_sh, ridx_v, m_v):
    c = lax.axis_index("c")
    s = lax.axis_index("s")
    wid = c * NS + s
    base = wid * EPW_S

    def zrow(i, _):
        for k in range(D // 16):
            m_v[i, pl.ds(k * 16, 16)] = jnp.zeros((16,), jnp.float32)
        return 0

    lax.fori_loop(0, CH, zrow, 0)

    def zchunk(k, _):
        j = s + k * NS

        @pl.when(j < N_ROW_CH)
        def _():
            pltpu.sync_copy(m_v, agg_sh.at[pl.ds(j * ROW_CH, ROW_CH)])
        return 0

    lax.fori_loop(0, (N_ROW_CH + NS - 1) // NS, zchunk, 0)
    plsc.subcore_barrier()

    def chunk(j, _):
        off = base + j * CH
        pltpu.sync_copy(recv_hbm.at[pl.ds(off, CH)], ridx_v)
        pltpu.sync_copy(m_hbm.at[pl.ds(off, CH)], m_v)
        pltpu.sync_copy(m_v, agg_sh.at[ridx_v], add=True)
        return 0

    lax.fori_loop(0, NCH, chunk, 0)
    plsc.subcore_barrier()

    def drain(k, _):
        j = s + k * NS

        @pl.when(j < N_ROW_CH)
        def _():
            pltpu.sync_copy(agg_sh.at[pl.ds(j * ROW_CH, ROW_CH)], m_v)
            pltpu.sync_copy(m_v, out_hbm.at[c, pl.ds(j * ROW_CH, ROW_CH)])
        return 0

    lax.fori_loop(0, (N_ROW_CH + NS - 1) // NS, drain, 0)


def _segment_sum(receivers, m):
    return pl.kernel(
        _seg_body,
        out_type=jax.ShapeDtypeStruct((NC, N_NODES, D), jnp.float32),
        mesh=_sc_mesh(),
        scratch_types=[
            pltpu.VMEM_SHARED((N_NODES, D), jnp.float32),
            pltpu.VMEM((CH,), jnp.int32),
            pltpu.VMEM((CH, D), jnp.float32),
        ],
    )(receivers, m)


def _comb_body(nf_ref, *rest):
    agg_refs = rest[:NSLICE]
    wt_ref, wb_ref, bc_ref, o_ref = rest[NSLICE:]
    agg = agg_refs[0][0] + agg_refs[0][1]
    for a in agg_refs[1:]:
        agg = agg + a[0] + a[1]
    o_ref[...] = (
        jnp.dot(nf_ref[...], wt_ref[...], preferred_element_type=jnp.float32)
        + jnp.dot(agg, wb_ref[...], preferred_element_type=jnp.float32)
        + bc_ref[...])


def _combine(node_feats, aggs, W_comb, b_comb):
    return pl.pallas_call(
        _comb_body,
        grid=(N_NBLK,),
        in_specs=[
            pl.BlockSpec((NODE_BLK, D), lambda i: (i, 0)),
            *[pl.BlockSpec((NC, NODE_BLK, D), lambda i: (0, i, 0))
              for _ in range(NSLICE)],
            pl.BlockSpec((D, D), lambda i: (0, 0)),
            pl.BlockSpec((D, D), lambda i: (0, 0)),
            pl.BlockSpec((1, D), lambda i: (0, 0)),
        ],
        out_specs=pl.BlockSpec((NODE_BLK, D), lambda i: (i, 0)),
        out_shape=jax.ShapeDtypeStruct((N_NODES, D), jnp.float32),
    )(node_feats, *aggs, W_comb[:D], W_comb[D:], b_comb.reshape(1, D))


def kernel(node_z, node_feats, senders, receivers, edge_weight, edge_feats,
           emb_table, W_dist, b_dist, W_comb, b_comb):
    node_z = node_z.astype(jnp.int32)
    senders = senders.astype(jnp.int32)
    receivers = receivers.astype(jnp.int32)
    z_j = _gather_zj(node_z, senders)
    aggs = []
    for k in range(NSLICE):
        lo, hi = k * E_SLICE, (k + 1) * E_SLICE
        m_k = _messages(z_j[lo:hi], edge_weight[lo:hi], edge_feats[lo:hi],
                        emb_table, W_dist, b_dist)
        aggs.append(_segment_sum(receivers[lo:hi], m_k))
    return _combine(node_feats, aggs, W_comb, b_comb)


# speedup vs baseline: 4.2638x; 1.0364x over previous
import functools

import jax
import jax.numpy as jnp
from jax import lax
from jax.experimental import pallas as pl
from jax.experimental.pallas import tpu as pltpu
from jax.experimental.pallas import tpu_sc as plsc

N_NODES = 10000
N_EDGES = 320000
D = 128
D_EDGE = 16
NUM_SPECIES = 100
CUTOFF = 5.0

NC = 2
NS = 16
NW = NC * NS
EPW = N_EDGES // NW
NSLICE = 2
E_SLICE = N_EDGES // NSLICE
DH = D // NC
EPT = E_SLICE // NS
CH = 1000
NCH = EPT // CH
ROW_CH = 1000
N_ROW_CH = N_NODES // ROW_CH

E_BLK = 1600
N_EBLK = N_EDGES // E_BLK

NODE_BLK = 1000
N_NBLK = N_NODES // NODE_BLK


def _sc_mesh():
    return plsc.VectorSubcoreMesh(
        core_axis_name="c", subcore_axis_name="s", num_cores=NC,
        num_subcores=NS)


def _zj_body(node_z_hbm, senders_hbm, zj_hbm, nz_v, snd_v, out_v):
    c = lax.axis_index("c")
    s = lax.axis_index("s")
    wid = c * NS + s
    base = wid * EPW
    pltpu.sync_copy(node_z_hbm, nz_v)
    pltpu.sync_copy(senders_hbm.at[pl.ds(base, EPW)], snd_v)

    def body(i, _):
        idx = snd_v[pl.ds(i * 16, 16)]
        out_v[pl.ds(i * 16, 16)] = plsc.load_gather(nz_v, [idx])
        return 0

    lax.fori_loop(0, EPW // 16, body, 0)
    pltpu.sync_copy(out_v, zj_hbm.at[pl.ds(base, EPW)])


def _gather_zj(node_z, senders):
    return pl.kernel(
        _zj_body,
        out_type=jax.ShapeDtypeStruct((N_EDGES,), jnp.int32),
        mesh=_sc_mesh(),
        scratch_types=[
            pltpu.VMEM((N_NODES,), jnp.int32),
            pltpu.VMEM((EPW,), jnp.int32),
            pltpu.VMEM((EPW,), jnp.int32),
        ],
        compiler_params=pltpu.CompilerParams(needs_layout_passes=False),
    )(node_z, senders)


def _msg_body(z_ref, ew_ref, ef_ref, emb_ref, wd_ref, bd_ref, m_ref):
    z = z_ref[0, 0]
    ew = ew_ref[0, 0]
    ef = ef_ref[...]
    cvals = 0.5 * (jnp.cos(ew * (jnp.pi / CUTOFF)) + 1.0)
    cvals = jnp.where(ew < CUTOFF, cvals, 0.0)
    onehot = (lax.broadcasted_iota(jnp.int32, (E_BLK, NUM_SPECIES), 1)
              == z[:, None]).astype(jnp.float32)
    xj = jnp.dot(onehot, emb_ref[...], preferred_element_type=jnp.float32)
    p = jnp.dot(ef, wd_ref[...], preferred_element_type=jnp.float32)
    p = p + bd_ref[...]
    m_ref[...] = xj * p * cvals[:, None]


def _messages(z_j, edge_weight, edge_feats, emb_table, W_dist, b_dist):
    n_blk = z_j.shape[0] // E_BLK
    return pl.pallas_call(
        _msg_body,
        grid=(n_blk,),
        in_specs=[
            pl.BlockSpec((1, 1, E_BLK), lambda i: (i, 0, 0)),
            pl.BlockSpec((1, 1, E_BLK), lambda i: (i, 0, 0)),
            pl.BlockSpec((E_BLK, D_EDGE), lambda i: (i, 0)),
            pl.BlockSpec((NUM_SPECIES, D), lambda i: (0, 0)),
            pl.BlockSpec((D_EDGE, D), lambda i: (0, 0)),
            pl.BlockSpec((1, D), lambda i: (0, 0)),
        ],
        out_specs=pl.BlockSpec((E_BLK, D), lambda i: (i, 0)),
        out_shape=jax.ShapeDtypeStruct((z_j.shape[0], D), jnp.float32),
    )(z_j.reshape(n_blk, 1, E_BLK), edge_weight.reshape(n_blk, 1, E_BLK),
      edge_feats, emb_table, W_dist, b_dist.reshape(1, D))


def _seg_body(recv_hbm, m_hbm, out_hbm, agg_sh, ridx_v, m_v):
    c = lax.axis_index("c")
    s = lax.axis_index("s")
    base = s * EPT
    col = c * DH

    def zrow(i, _):
        for k in range(DH // 16):
            m_v[i, pl.ds(k * 16, 16)] = jnp.zeros((16,), jnp.float32)
        return 0

    lax.fori_loop(0, ROW_CH, zrow, 0)

    def zchunk(k, _):
        j = s + k * NS

        @pl.when(j < N_ROW_CH)
        def _():
            pltpu.sync_copy(m_v, agg_sh.at[pl.ds(j * ROW_CH, ROW_CH)])
        return 0

    lax.fori_loop(0, (N_ROW_CH + NS - 1) // NS, zchunk, 0)
    plsc.subcore_barrier()

    def chunk(j, _):
        off = base + j * CH
        pltpu.sync_copy(recv_hbm.at[pl.ds(off, CH)], ridx_v)
        pltpu.sync_copy(m_hbm.at[pl.ds(off, CH), pl.ds(col, DH)], m_v)
        pltpu.sync_copy(m_v, agg_sh.at[ridx_v], add=True)
        return 0

    lax.fori_loop(0, NCH, chunk, 0)
    plsc.subcore_barrier()

    def drain(k, _):
        j = s + k * NS

        @pl.when(j < N_ROW_CH)
        def _():
            pltpu.sync_copy(agg_sh.at[pl.ds(j * ROW_CH, ROW_CH)], m_v)
            pltpu.sync_copy(
                m_v, out_hbm.at[pl.ds(j * ROW_CH, ROW_CH), pl.ds(col, DH)])
        return 0

    lax.fori_loop(0, (N_ROW_CH + NS - 1) // NS, drain, 0)


def _segment_sum(receivers, m):
    return pl.kernel(
        _seg_body,
        out_type=jax.ShapeDtypeStruct((N_NODES, D), jnp.float32),
        mesh=_sc_mesh(),
        scratch_types=[
            pltpu.VMEM_SHARED((N_NODES, DH), jnp.float32),
            pltpu.VMEM((CH,), jnp.int32),
            pltpu.VMEM((ROW_CH, DH), jnp.float32),
        ],
        compiler_params=pltpu.CompilerParams(use_tc_tiling_on_sc=False),
    )(receivers, m)


def _comb_body(nf_ref, *rest):
    agg_refs = rest[:NSLICE]
    wt_ref, wb_ref, bc_ref, o_ref = rest[NSLICE:]
    agg = agg_refs[0][...]
    for a in agg_refs[1:]:
        agg = agg + a[...]
    o_ref[...] = (
        jnp.dot(nf_ref[...], wt_ref[...], preferred_element_type=jnp.float32)
        + jnp.dot(agg, wb_ref[...], preferred_element_type=jnp.float32)
        + bc_ref[...])


def _combine(node_feats, aggs, W_comb, b_comb):
    return pl.pallas_call(
        _comb_body,
        grid=(N_NBLK,),
        in_specs=[
            pl.BlockSpec((NODE_BLK, D), lambda i: (i, 0)),
            *[pl.BlockSpec((NODE_BLK, D), lambda i: (i, 0))
              for _ in range(NSLICE)],
            pl.BlockSpec((D, D), lambda i: (0, 0)),
            pl.BlockSpec((D, D), lambda i: (0, 0)),
            pl.BlockSpec((1, D), lambda i: (0, 0)),
        ],
        out_specs=pl.BlockSpec((NODE_BLK, D), lambda i: (i, 0)),
        out_shape=jax.ShapeDtypeStruct((N_NODES, D), jnp.float32),
    )(node_feats, *aggs, W_comb[:D], W_comb[D:], b_comb.reshape(1, D))


def kernel(node_z, node_feats, senders, receivers, edge_weight, edge_feats,
           emb_table, W_dist, b_dist, W_comb, b_comb):
    node_z = node_z.astype(jnp.int32)
    senders = senders.astype(jnp.int32)
    receivers = receivers.astype(jnp.int32)
    z_j = _gather_zj(node_z, senders)
    aggs = []
    for k in range(NSLICE):
        lo, hi = k * E_SLICE, (k + 1) * E_SLICE
        m_k = _messages(z_j[lo:hi], edge_weight[lo:hi], edge_feats[lo:hi],
                        emb_table, W_dist, b_dist)
        aggs.append(_segment_sum(receivers[lo:hi], m_k))
    return _combine(node_feats, aggs, W_comb, b_comb)


# speedup vs baseline: 4.4326x; 1.0396x over previous
import functools

import jax
import jax.numpy as jnp
from jax import lax
from jax.experimental import pallas as pl
from jax.experimental.pallas import tpu as pltpu
from jax.experimental.pallas import tpu_sc as plsc

N_NODES = 10000
N_EDGES = 320000
D = 128
D_EDGE = 16
NUM_SPECIES = 100
CUTOFF = 5.0

NC = 2
NS = 16
NW = NC * NS
EPW = N_EDGES // NW
NSLICE = 2
E_SLICE = N_EDGES // NSLICE
EPW_Z = E_SLICE // NW
DH = D // NC
EPT = E_SLICE // NS
CH = 1000
NCH = EPT // CH
ROW_CH = 1000
N_ROW_CH = N_NODES // ROW_CH

E_BLK = 1600
N_EBLK = N_EDGES // E_BLK

NODE_BLK = 1000
N_NBLK = N_NODES // NODE_BLK


def _sc_mesh():
    return plsc.VectorSubcoreMesh(
        core_axis_name="c", subcore_axis_name="s", num_cores=NC,
        num_subcores=NS)


def _zj_body(slice_base, node_z_hbm, senders_hbm, zj_hbm, nz_v, snd_v, out_v):
    c = lax.axis_index("c")
    s = lax.axis_index("s")
    wid = c * NS + s
    base = slice_base + wid * EPW_Z
    pltpu.sync_copy(node_z_hbm, nz_v)
    pltpu.sync_copy(senders_hbm.at[pl.ds(base, EPW_Z)],
                    snd_v.at[pl.ds(0, EPW_Z)])

    def body(i, _):
        idx = snd_v[pl.ds(i * 16, 16)]
        idx = jnp.maximum(jnp.minimum(idx, N_NODES - 1), 0)
        out_v[pl.ds(i * 16, 16)] = plsc.load_gather(nz_v, [idx])
        return 0

    lax.fori_loop(0, (EPW_Z + 15) // 16, body, 0)
    pltpu.sync_copy(out_v.at[pl.ds(0, EPW_Z)],
                    zj_hbm.at[pl.ds(wid * EPW_Z, EPW_Z)])


def _gather_zj(node_z, senders, slice_base):
    return pl.kernel(
        functools.partial(_zj_body, slice_base),
        out_type=jax.ShapeDtypeStruct((E_SLICE,), jnp.int32),
        mesh=_sc_mesh(),
        scratch_types=[
            pltpu.VMEM((N_NODES,), jnp.int32),
            pltpu.VMEM(((EPW_Z + 15) // 16 * 16,), jnp.int32),
            pltpu.VMEM(((EPW_Z + 15) // 16 * 16,), jnp.int32),
        ],
        compiler_params=pltpu.CompilerParams(needs_layout_passes=False),
    )(node_z, senders)


def _msg_body(z_ref, ew_ref, ef_ref, emb_ref, wd_ref, bd_ref, m_ref):
    z = z_ref[0, 0]
    ew = ew_ref[0, 0]
    ef = ef_ref[...]
    cvals = 0.5 * (jnp.cos(ew * (jnp.pi / CUTOFF)) + 1.0)
    cvals = jnp.where(ew < CUTOFF, cvals, 0.0)
    onehot = (lax.broadcasted_iota(jnp.int32, (E_BLK, NUM_SPECIES), 1)
              == z[:, None]).astype(jnp.float32)
    xj = jnp.dot(onehot, emb_ref[...], preferred_element_type=jnp.float32)
    p = jnp.dot(ef, wd_ref[...], preferred_element_type=jnp.float32)
    p = p + bd_ref[...]
    m_ref[...] = xj * p * cvals[:, None]


def _messages(z_j3, ew3, edge_feats, emb_table, W_dist, b_dist, blk0):
    n_blk = E_SLICE // E_BLK
    return pl.pallas_call(
        _msg_body,
        grid=(n_blk,),
        in_specs=[
            pl.BlockSpec((1, 1, E_BLK), lambda i: (i, 0, 0)),
            pl.BlockSpec((1, 1, E_BLK), lambda i: (i + blk0, 0, 0)),
            pl.BlockSpec((E_BLK, D_EDGE), lambda i: (i + blk0, 0)),
            pl.BlockSpec((NUM_SPECIES, D), lambda i: (0, 0)),
            pl.BlockSpec((D_EDGE, D), lambda i: (0, 0)),
            pl.BlockSpec((1, D), lambda i: (0, 0)),
        ],
        out_specs=pl.BlockSpec((E_BLK, D), lambda i: (i, 0)),
        out_shape=jax.ShapeDtypeStruct((E_SLICE, D), jnp.float32),
    )(z_j3, ew3, edge_feats, emb_table, W_dist, b_dist.reshape(1, D))


def _seg_body(slice_base, recv_hbm, m_hbm, out_hbm, agg_sh, ridx_v, m_v):
    c = lax.axis_index("c")
    s = lax.axis_index("s")
    base = s * EPT
    col = c * DH

    def zrow(i, _):
        for k in range(DH // 16):
            m_v[i, pl.ds(k * 16, 16)] = jnp.zeros((16,), jnp.float32)
        return 0

    lax.fori_loop(0, ROW_CH, zrow, 0)

    def zchunk(k, _):
        j = s + k * NS

        @pl.when(j < N_ROW_CH)
        def _():
            pltpu.sync_copy(m_v, agg_sh.at[pl.ds(j * ROW_CH, ROW_CH)])
        return 0

    lax.fori_loop(0, (N_ROW_CH + NS - 1) // NS, zchunk, 0)
    plsc.subcore_barrier()

    def chunk(j, _):
        off = base + j * CH
        pltpu.sync_copy(recv_hbm.at[pl.ds(slice_base + off, CH)], ridx_v)
        pltpu.sync_copy(m_hbm.at[pl.ds(off, CH), pl.ds(col, DH)], m_v)
        pltpu.sync_copy(m_v, agg_sh.at[ridx_v], add=True)
        return 0

    lax.fori_loop(0, NCH, chunk, 0)
    plsc.subcore_barrier()

    def drain(k, _):
        j = s + k * NS

        @pl.when(j < N_ROW_CH)
        def _():
            pltpu.sync_copy(agg_sh.at[pl.ds(j * ROW_CH, ROW_CH)], m_v)
            pltpu.sync_copy(
                m_v, out_hbm.at[pl.ds(j * ROW_CH, ROW_CH), pl.ds(col, DH)])
        return 0

    lax.fori_loop(0, (N_ROW_CH + NS - 1) // NS, drain, 0)


def _segment_sum(receivers, m, slice_base):
    return pl.kernel(
        functools.partial(_seg_body, slice_base),
        out_type=jax.ShapeDtypeStruct((N_NODES, D), jnp.float32),
        mesh=_sc_mesh(),
        scratch_types=[
            pltpu.VMEM_SHARED((N_NODES, DH), jnp.float32),
            pltpu.VMEM((CH,), jnp.int32),
            pltpu.VMEM((ROW_CH, DH), jnp.float32),
        ],
        compiler_params=pltpu.CompilerParams(use_tc_tiling_on_sc=False),
    )(receivers, m)


def _comb_body(nf_ref, *rest):
    agg_refs = rest[:NSLICE]
    wt_ref, wb_ref, bc_ref, o_ref = rest[NSLICE:]
    agg = agg_refs[0][...]
    for a in agg_refs[1:]:
        agg = agg + a[...]
    o_ref[...] = (
        jnp.dot(nf_ref[...], wt_ref[...], preferred_element_type=jnp.float32)
        + jnp.dot(agg, wb_ref[...], preferred_element_type=jnp.float32)
        + bc_ref[...])


def _combine(node_feats, aggs, W_comb, b_comb):
    return pl.pallas_call(
        _comb_body,
        grid=(N_NBLK,),
        in_specs=[
            pl.BlockSpec((NODE_BLK, D), lambda i: (i, 0)),
            *[pl.BlockSpec((NODE_BLK, D), lambda i: (i, 0))
              for _ in range(NSLICE)],
            pl.BlockSpec((D, D), lambda i: (0, 0)),
            pl.BlockSpec((D, D), lambda i: (0, 0)),
            pl.BlockSpec((1, D), lambda i: (0, 0)),
        ],
        out_specs=pl.BlockSpec((NODE_BLK, D), lambda i: (i, 0)),
        out_shape=jax.ShapeDtypeStruct((N_NODES, D), jnp.float32),
    )(node_feats, *aggs, W_comb[:D], W_comb[D:], b_comb.reshape(1, D))


def kernel(node_z, node_feats, senders, receivers, edge_weight, edge_feats,
           emb_table, W_dist, b_dist, W_comb, b_comb):
    node_z = node_z.astype(jnp.int32)
    senders = senders.astype(jnp.int32)
    receivers = receivers.astype(jnp.int32)
    ew3 = edge_weight.reshape(N_EBLK, 1, E_BLK)
    n_blk = E_SLICE // E_BLK
    zjs = [_gather_zj(node_z, senders, k * E_SLICE) for k in range(NSLICE)]
    aggs = []
    for k in range(NSLICE):
        m_k = _messages(zjs[k].reshape(n_blk, 1, E_BLK), ew3, edge_feats,
                        emb_table, W_dist, b_dist, k * n_blk)
        aggs.append(_segment_sum(receivers, m_k, k * E_SLICE))
    return _combine(node_feats, aggs, W_comb, b_comb)


# speedup vs baseline: 4.4536x; 1.0047x over previous
import functools

import jax
import jax.numpy as jnp
from jax import lax
from jax.experimental import pallas as pl
from jax.experimental.pallas import tpu as pltpu
from jax.experimental.pallas import tpu_sc as plsc

N_NODES = 10000
N_EDGES = 320000
D = 128
D_EDGE = 16
NUM_SPECIES = 100
CUTOFF = 5.0

NC = 2
NS = 16
NW = NC * NS
EPW = N_EDGES // NW
NSLICE = 2
E_SLICE = N_EDGES // NSLICE
EPW_Z = E_SLICE // NW
DH = D // NC
EPT = E_SLICE // NS
CH = 1000
NCH = EPT // CH
ROW_CH = 1000
N_ROW_CH = N_NODES // ROW_CH

E_BLK = 1600
N_EBLK = N_EDGES // E_BLK

NODE_BLK = 1000
N_NBLK = N_NODES // NODE_BLK


def _sc_mesh():
    return plsc.VectorSubcoreMesh(
        core_axis_name="c", subcore_axis_name="s", num_cores=NC,
        num_subcores=NS)


def _zj_body(node_z_hbm, senders_hbm, zj_hbm, nz_v, snd_v, out_v):
    c = lax.axis_index("c")
    s = lax.axis_index("s")
    wid = c * NS + s
    base = wid * EPW
    pltpu.sync_copy(node_z_hbm, nz_v)
    pltpu.sync_copy(senders_hbm.at[pl.ds(base, EPW)], snd_v)

    def body(i, _):
        idx = snd_v[pl.ds(i * 16, 16)]
        out_v[pl.ds(i * 16, 16)] = plsc.load_gather(nz_v, [idx])
        return 0

    lax.fori_loop(0, EPW // 16, body, 0)
    pltpu.sync_copy(out_v, zj_hbm.at[pl.ds(base, EPW)])


def _gather_zj(node_z, senders):
    return pl.kernel(
        _zj_body,
        out_type=jax.ShapeDtypeStruct((N_EDGES,), jnp.int32),
        mesh=_sc_mesh(),
        scratch_types=[
            pltpu.VMEM((N_NODES,), jnp.int32),
            pltpu.VMEM((EPW,), jnp.int32),
            pltpu.VMEM((EPW,), jnp.int32),
        ],
        compiler_params=pltpu.CompilerParams(needs_layout_passes=False),
    )(node_z, senders)


def _msg_body(z_ref, ew_ref, ef_ref, emb_ref, wd_ref, bd_ref, m_ref):
    z = z_ref[0, 0]
    ew = ew_ref[0, 0]
    ef = ef_ref[...]
    cvals = 0.5 * (jnp.cos(ew * (jnp.pi / CUTOFF)) + 1.0)
    cvals = jnp.where(ew < CUTOFF, cvals, 0.0)
    onehot = (lax.broadcasted_iota(jnp.int32, (E_BLK, NUM_SPECIES), 1)
              == z[:, None]).astype(jnp.float32)
    xj = jnp.dot(onehot, emb_ref[...], preferred_element_type=jnp.float32)
    p = jnp.dot(ef, wd_ref[...], preferred_element_type=jnp.float32)
    p = p + bd_ref[...]
    m_ref[...] = xj * p * cvals[:, None]


def _messages(z_j3, ew3, edge_feats, emb_table, W_dist, b_dist, blk0):
    n_blk = E_SLICE // E_BLK
    return pl.pallas_call(
        _msg_body,
        grid=(n_blk,),
        in_specs=[
            pl.BlockSpec((1, 1, E_BLK), lambda i: (i + blk0, 0, 0)),
            pl.BlockSpec((1, 1, E_BLK), lambda i: (i + blk0, 0, 0)),
            pl.BlockSpec((E_BLK, D_EDGE), lambda i: (i + blk0, 0)),
            pl.BlockSpec((NUM_SPECIES, D), lambda i: (0, 0)),
            pl.BlockSpec((D_EDGE, D), lambda i: (0, 0)),
            pl.BlockSpec((1, D), lambda i: (0, 0)),
        ],
        out_specs=pl.BlockSpec((E_BLK, D), lambda i: (i, 0)),
        out_shape=jax.ShapeDtypeStruct((E_SLICE, D), jnp.float32),
    )(z_j3, ew3, edge_feats, emb_table, W_dist, b_dist.reshape(1, D))


def _seg_body(slice_base, recv_hbm, m_hbm, out_hbm, agg_sh, ridx_v, m_v):
    c = lax.axis_index("c")
    s = lax.axis_index("s")
    base = s * EPT
    col = c * DH

    def zrow(i, _):
        for k in range(DH // 16):
            m_v[i, pl.ds(k * 16, 16)] = jnp.zeros((16,), jnp.float32)
        return 0

    lax.fori_loop(0, ROW_CH, zrow, 0)

    def zchunk(k, _):
        j = s + k * NS

        @pl.when(j < N_ROW_CH)
        def _():
            pltpu.sync_copy(m_v, agg_sh.at[pl.ds(j * ROW_CH, ROW_CH)])
        return 0

    lax.fori_loop(0, (N_ROW_CH + NS - 1) // NS, zchunk, 0)
    plsc.subcore_barrier()

    def chunk(j, _):
        off = base + j * CH
        pltpu.sync_copy(recv_hbm.at[pl.ds(slice_base + off, CH)], ridx_v)
        pltpu.sync_copy(m_hbm.at[pl.ds(off, CH), pl.ds(col, DH)], m_v)
        pltpu.sync_copy(m_v, agg_sh.at[ridx_v], add=True)
        return 0

    lax.fori_loop(0, NCH, chunk, 0)
    plsc.subcore_barrier()

    def drain(k, _):
        j = s + k * NS

        @pl.when(j < N_ROW_CH)
        def _():
            pltpu.sync_copy(agg_sh.at[pl.ds(j * ROW_CH, ROW_CH)], m_v)
            pltpu.sync_copy(
                m_v, out_hbm.at[pl.ds(j * ROW_CH, ROW_CH), pl.ds(col, DH)])
        return 0

    lax.fori_loop(0, (N_ROW_CH + NS - 1) // NS, drain, 0)


def _segment_sum(receivers, m, slice_base):
    return pl.kernel(
        functools.partial(_seg_body, slice_base),
        out_type=jax.ShapeDtypeStruct((N_NODES, D), jnp.float32),
        mesh=_sc_mesh(),
        scratch_types=[
            pltpu.VMEM_SHARED((N_NODES, DH), jnp.float32),
            pltpu.VMEM((CH,), jnp.int32),
            pltpu.VMEM((ROW_CH, DH), jnp.float32),
        ],
        compiler_params=pltpu.CompilerParams(use_tc_tiling_on_sc=False),
    )(receivers, m)


def _comb_body(nf_ref, *rest):
    agg_refs = rest[:NSLICE]
    wt_ref, wb_ref, bc_ref, o_ref = rest[NSLICE:]
    agg = agg_refs[0][...]
    for a in agg_refs[1:]:
        agg = agg + a[...]
    o_ref[...] = (
        jnp.dot(nf_ref[...], wt_ref[...], preferred_element_type=jnp.float32)
        + jnp.dot(agg, wb_ref[...], preferred_element_type=jnp.float32)
        + bc_ref[...])


def _combine(node_feats, aggs, W_comb, b_comb):
    return pl.pallas_call(
        _comb_body,
        grid=(N_NBLK,),
        in_specs=[
            pl.BlockSpec((NODE_BLK, D), lambda i: (i, 0)),
            *[pl.BlockSpec((NODE_BLK, D), lambda i: (i, 0))
              for _ in range(NSLICE)],
            pl.BlockSpec((D, D), lambda i: (0, 0)),
            pl.BlockSpec((D, D), lambda i: (0, 0)),
            pl.BlockSpec((1, D), lambda i: (0, 0)),
        ],
        out_specs=pl.BlockSpec((NODE_BLK, D), lambda i: (i, 0)),
        out_shape=jax.ShapeDtypeStruct((N_NODES, D), jnp.float32),
    )(node_feats, *aggs, W_comb[:D], W_comb[D:], b_comb.reshape(1, D))


def kernel(node_z, node_feats, senders, receivers, edge_weight, edge_feats,
           emb_table, W_dist, b_dist, W_comb, b_comb):
    node_z = node_z.astype(jnp.int32)
    senders = senders.astype(jnp.int32)
    receivers = receivers.astype(jnp.int32)
    ew3 = edge_weight.reshape(N_EBLK, 1, E_BLK)
    n_blk = E_SLICE // E_BLK
    zj3 = _gather_zj(node_z, senders).reshape(N_EBLK, 1, E_BLK)
    aggs = []
    for k in range(NSLICE):
        m_k = _messages(zj3, ew3, edge_feats,
                        emb_table, W_dist, b_dist, k * n_blk)
        aggs.append(_segment_sum(receivers, m_k, k * E_SLICE))
    return _combine(node_feats, aggs, W_comb, b_comb)
